# baseline (speedup 1.0000x reference)
import jax
import jax.numpy as jnp
import numpy as np
from jax.experimental import pallas as pl
from jax.experimental.pallas import tpu as pltpu

_EPS = 1e-06
_S = 0.025
_ALPHA = 0.98
_DELTA = 2.0
_R = 0.5

_TC = 512
_BB = 2


def _pcen_kernel(x_ref, l_ref, o_ref):
    F = x_ref.shape[1]
    T = x_ref.shape[2]
    lmat = l_ref[...]
    decay = lmat[0:1, :] * ((1.0 - _S) / _S)
    lmat_bf = lmat.astype(jnp.bfloat16)
    sqrt_delta = np.float32(np.sqrt(_DELTA))

    for b in range(_BB):
        carry = jnp.zeros((F, 1), jnp.float32)
        for c in range(T // _TC):
            x = x_ref[b, :, pl.ds(c * _TC, _TC)]
            y = jax.lax.dot_general(
                x.astype(jnp.bfloat16),
                lmat_bf,
                (((1,), (0,)), ((), ())),
                preferred_element_type=jnp.float32,
            )
            m = y + carry * decay
            carry = m[:, _TC - 1 : _TC]
            u = x * jnp.exp2(-_ALPHA * jnp.log2(m + _EPS)) + _DELTA
            o_ref[b, :, pl.ds(c * _TC, _TC)] = u * jax.lax.rsqrt(u) - sqrt_delta


@jax.jit
def kernel(x):
    B, F, T = x.shape
    j = np.arange(_TC)
    diff = j[None, :] - j[:, None]
    L = np.where(diff >= 0, _S * (1.0 - _S) ** diff, 0.0)
    L = jnp.asarray(L, dtype=jnp.float32)

    return pl.pallas_call(
        _pcen_kernel,
        grid=(B // _BB,),
        in_specs=[
            pl.BlockSpec((_BB, F, T), lambda b: (b, 0, 0)),
            pl.BlockSpec((_TC, _TC), lambda b: (0, 0)),
        ],
        out_specs=pl.BlockSpec((_BB, F, T), lambda b: (b, 0, 0)),
        out_shape=jax.ShapeDtypeStruct((B, F, T), jnp.float32),
        compiler_params=pltpu.CompilerParams(
            dimension_semantics=("parallel",)
        ),
    )(x, L)


# speedup vs baseline: 1.0310x; 1.0310x over previous
import jax
import jax.numpy as jnp
import numpy as np
from jax.experimental import pallas as pl
from jax.experimental.pallas import tpu as pltpu

_EPS = 1e-06
_S = 0.025
_ALPHA = 0.98
_DELTA = 2.0
_R = 0.5

_TC = 128
_BB = 2


def _pcen_kernel(x_ref, l_ref, o_ref):
    F = x_ref.shape[1]
    T = x_ref.shape[2]
    lmat = l_ref[...]
    decay = lmat[0:1, :] * ((1.0 - _S) / _S)
    lmat_bf = lmat.astype(jnp.bfloat16)
    sqrt_delta = np.float32(np.sqrt(_DELTA))

    for b in range(_BB):
        carry = jnp.zeros((F, 1), jnp.float32)
        for c in range(T // _TC):
            x = x_ref[b, :, pl.ds(c * _TC, _TC)]
            y = jax.lax.dot_general(
                x.astype(jnp.bfloat16),
                lmat_bf,
                (((1,), (0,)), ((), ())),
                preferred_element_type=jnp.float32,
            )
            m = y + carry * decay
            carry = m[:, _TC - 1 : _TC]
            u = x * jnp.exp2(-_ALPHA * jnp.log2(m + _EPS)) + _DELTA
            o_ref[b, :, pl.ds(c * _TC, _TC)] = u * jax.lax.rsqrt(u) - sqrt_delta


@jax.jit
def kernel(x):
    B, F, T = x.shape
    j = np.arange(_TC)
    diff = j[None, :] - j[:, None]
    L = np.where(diff >= 0, _S * (1.0 - _S) ** diff, 0.0)
    L = jnp.asarray(L, dtype=jnp.float32)

    return pl.pallas_call(
        _pcen_kernel,
        grid=(B // _BB,),
        in_specs=[
            pl.BlockSpec((_BB, F, T), lambda b: (b, 0, 0)),
            pl.BlockSpec((_TC, _TC), lambda b: (0, 0)),
        ],
        out_specs=pl.BlockSpec((_BB, F, T), lambda b: (b, 0, 0)),
        out_shape=jax.ShapeDtypeStruct((B, F, T), jnp.float32),
        compiler_params=pltpu.CompilerParams(
            dimension_semantics=("parallel",)
        ),
    )(x, L)


# speedup vs baseline: 1.0531x; 1.0214x over previous
import jax
import jax.numpy as jnp
import numpy as np
from jax.experimental import pallas as pl
from jax.experimental.pallas import tpu as pltpu

_EPS = 1e-06
_S = 0.025
_ALPHA = 0.98
_DELTA = 2.0
_R = 0.5

_TC = 256
_BB = 2


def _pcen_kernel(x_ref, l_ref, o_ref):
    F = x_ref.shape[1]
    T = x_ref.shape[2]
    lmat = l_ref[...]
    decay = lmat[0:1, :] * ((1.0 - _S) / _S)
    lmat_bf = lmat.astype(jnp.bfloat16)
    sqrt_delta = np.float32(np.sqrt(_DELTA))

    for b in range(_BB):
        carry = jnp.zeros((F, 1), jnp.float32)
        for c in range(T // _TC):
            x = x_ref[b, :, pl.ds(c * _TC, _TC)]
            y = jax.lax.dot_general(
                x.astype(jnp.bfloat16),
                lmat_bf,
                (((1,), (0,)), ((), ())),
                preferred_element_type=jnp.float32,
            )
            m = y + carry * decay
            carry = m[:, _TC - 1 : _TC]
            u = x * jnp.exp2(-_ALPHA * jnp.log2(m + _EPS)) + _DELTA
            o_ref[b, :, pl.ds(c * _TC, _TC)] = u * jax.lax.rsqrt(u) - sqrt_delta


@jax.jit
def kernel(x):
    B, F, T = x.shape
    j = np.arange(_TC)
    diff = j[None, :] - j[:, None]
    L = np.where(diff >= 0, _S * (1.0 - _S) ** diff, 0.0)
    L = jnp.asarray(L, dtype=jnp.float32)

    return pl.pallas_call(
        _pcen_kernel,
        grid=(B // _BB,),
        in_specs=[
            pl.BlockSpec((_BB, F, T), lambda b: (b, 0, 0)),
            pl.BlockSpec((_TC, _TC), lambda b: (0, 0)),
        ],
        out_specs=pl.BlockSpec((_BB, F, T), lambda b: (b, 0, 0)),
        out_shape=jax.ShapeDtypeStruct((B, F, T), jnp.float32),
        compiler_params=pltpu.CompilerParams(
            dimension_semantics=("parallel",)
        ),
    )(x, L)


# speedup vs baseline: 1.0956x; 1.0404x over previous
import jax
import jax.numpy as jnp
import numpy as np
from jax.experimental import pallas as pl
from jax.experimental.pallas import tpu as pltpu

_EPS = 1e-06
_S = 0.025
_ALPHA = 0.98
_DELTA = 2.0
_R = 0.5

_TC = 256
_BB = 2
_NLOAD = 3


def _pcen_kernel(x_hbm, l_ref, o_hbm, xbuf, obuf, lsem, ssem):
    F = x_hbm.shape[1]
    T = x_hbm.shape[2]
    g = pl.program_id(0)
    ng = pl.num_programs(0)
    lslot = jax.lax.rem(g, _NLOAD)
    sslot = jax.lax.rem(g, 2)

    def load_copy(step):
        return pltpu.make_async_copy(
            x_hbm.at[pl.ds(step * _BB, _BB)],
            xbuf.at[jax.lax.rem(step, _NLOAD)],
            lsem.at[jax.lax.rem(step, _NLOAD)],
        )

    def store_copy(step):
        return pltpu.make_async_copy(
            obuf.at[jax.lax.rem(step, 2)],
            o_hbm.at[pl.ds(step * _BB, _BB)],
            ssem.at[jax.lax.rem(step, 2)],
        )

    @pl.when(g == 0)
    def _():
        for p in range(_NLOAD):
            load_copy(p).start()

    load_copy(g).wait()

    @pl.when(g >= 2)
    def _():
        store_copy(g - 2).wait()

    lmat = l_ref[...]
    decay = lmat[0:1, :] * ((1.0 - _S) / _S)
    lmat_bf = lmat.astype(jnp.bfloat16)
    sqrt_delta = np.float32(np.sqrt(_DELTA))

    for b in range(_BB):
        carry = jnp.zeros((F, 1), jnp.float32)
        for c in range(T // _TC):
            x = xbuf[lslot, b, :, pl.ds(c * _TC, _TC)]
            y = jax.lax.dot_general(
                x.astype(jnp.bfloat16),
                lmat_bf,
                (((1,), (0,)), ((), ())),
                preferred_element_type=jnp.float32,
            )
            m = y + carry * decay
            carry = m[:, _TC - 1 : _TC]
            u = x * jnp.exp2(-_ALPHA * jnp.log2(m + _EPS)) + _DELTA
            obuf[sslot, b, :, pl.ds(c * _TC, _TC)] = (
                u * jax.lax.rsqrt(u) - sqrt_delta
            )

    store_copy(g).start()

    @pl.when(g < ng - _NLOAD)
    def _():
        load_copy(g + _NLOAD).start()

    @pl.when(g == ng - 1)
    def _():
        store_copy(g - 1).wait()
        store_copy(g).wait()


@jax.jit
def kernel(x):
    B, F, T = x.shape
    j = np.arange(_TC)
    diff = j[None, :] - j[:, None]
    L = np.where(diff >= 0, _S * (1.0 - _S) ** diff, 0.0)
    L = jnp.asarray(L, dtype=jnp.float32)

    return pl.pallas_call(
        _pcen_kernel,
        grid=(B // _BB,),
        in_specs=[
            pl.BlockSpec(memory_space=pltpu.MemorySpace.HBM),
            pl.BlockSpec((_TC, _TC), lambda b: (0, 0)),
        ],
        out_specs=pl.BlockSpec(memory_space=pltpu.MemorySpace.HBM),
        out_shape=jax.ShapeDtypeStruct((B, F, T), jnp.float32),
        scratch_shapes=[
            pltpu.VMEM((_NLOAD, _BB, F, T), jnp.float32),
            pltpu.VMEM((2, _BB, F, T), jnp.float32),
            pltpu.SemaphoreType.DMA((_NLOAD,)),
            pltpu.SemaphoreType.DMA((2,)),
        ],
        compiler_params=pltpu.CompilerParams(
            dimension_semantics=("arbitrary",)
        ),
    )(x, L)
